# baseline (speedup 1.0000x reference)
import functools

import jax
import jax.numpy as jnp
from jax import lax
from jax.experimental import pallas as pl
from jax.experimental.pallas import tpu as pltpu
from jax.experimental.pallas import tpu_sc as plsc

N = 10000
E = 320000
D = 128
OUT = 128
EPS = 1e-07
N_HEADS = 6

NC = 2
NS = 16
CHUNK = 128
NPAD = 64
EP = 327680
NROWS = EP // CHUNK
ROWS_TILE = NROWS // NS
NSPLIT = 640
NLAST = N - NSPLIT * (NS - 1)
STAGE = 40
RING = 2
NACC = N + NPAD


def _square_stack_body(x_ref, out_ref):
    xb = x_ref[...]
    out_ref[0] = xb
    out_ref[1] = xb * xb


def _square_stack(x):
    blk = 2000
    return pl.pallas_call(
        _square_stack_body,
        grid=(N // blk,),
        in_specs=[pl.BlockSpec((blk, D), lambda i: (i, 0))],
        out_specs=pl.BlockSpec((2, blk, D), lambda i: (0, i, 0)),
        out_shape=jax.ShapeDtypeStruct((2, N, D), jnp.float32),
    )(x)


def _seg_body(xs_hbm, row_hbm, col_hbm, colN_hbm, s1_hbm, s2_hbm, deg_hbm,
              row_v, col_v, gb0, gb1, zb, ones_v, acc, dacc,
              gsems, ssems, dsem):
    cid = lax.axis_index("c")
    sid = lax.axis_index("s")
    last = sid == NS - 1
    gbufs = (gb0, gb1)

    zero16 = jnp.zeros((16,), jnp.float32)
    one16 = jnp.full((16,), 1.0, jnp.float32)

    def _zg(t, _):
        r = t // (D // 16)
        k = (t % (D // 16)) * 16
        gb0[r, pl.ds(k, 16)] = zero16
        gb1[r, pl.ds(k, 16)] = one16
        return 0
    lax.fori_loop(0, CHUNK * (D // 16), _zg, 0)

    stg = sid * 2 * CHUNK
    pltpu.sync_copy(gb0.at[pl.ds(0, CHUNK), pl.ds(0, 8)], dacc.at[pl.ds(stg, CHUNK)])
    pltpu.sync_copy(gb1.at[pl.ds(0, CHUNK), pl.ds(0, 8)],
                    dacc.at[pl.ds(stg + CHUNK, CHUNK)])
    pltpu.sync_copy(dacc.at[pl.ds(stg, CHUNK)], zb)
    pltpu.sync_copy(dacc.at[pl.ds(stg + CHUNK, CHUNK)], ones_v)

    nbase = sid * NSPLIT
    nz = jnp.where(last, NLAST // CHUNK, NSPLIT // CHUNK)

    def _zacc(t, _):
        pltpu.sync_copy(gb0, acc.at[pl.ds(nbase + t * CHUNK, CHUNK)])
        pltpu.sync_copy(zb, dacc.at[pl.ds(nbase + t * CHUNK, CHUNK)])
        return 0
    lax.fori_loop(0, nz, _zacc, 0)

    @pl.when(last)
    def _():
        tb = nbase + (NLAST // CHUNK) * CHUNK
        tail = NLAST % CHUNK + NPAD
        pltpu.sync_copy(gb0.at[pl.ds(0, tail)], acc.at[pl.ds(tb, tail)])
        pltpu.sync_copy(zb.at[pl.ds(0, tail)], dacc.at[pl.ds(tb, tail)])

    plsc.subcore_barrier()

    nstages = ROWS_TILE // STAGE

    def _stage(st, _):
        rb = sid * ROWS_TILE + st * STAGE
        pltpu.sync_copy(row_hbm.at[pl.ds(rb, STAGE)], row_v)

        @pl.when(cid == 0)
        def _():
            pltpu.sync_copy(col_hbm.at[pl.ds(rb, STAGE)], col_v)

        @pl.when(cid == 1)
        def _():
            pltpu.sync_copy(colN_hbm.at[pl.ds(rb, STAGE)], col_v)

        def _ring(kk, _):
            for b in range(RING):
                j = kk * RING + b

                @pl.when(kk > 0)
                def _(b=b, j=j):
                    pltpu.make_async_copy(gbufs[b], acc.at[row_v.at[j]],
                                          ssems.at[b]).wait()
                pltpu.async_copy(xs_hbm.at[col_v.at[j]], gbufs[b], gsems.at[b])
            for b in range(RING):
                j = kk * RING + b
                pltpu.make_async_copy(xs_hbm.at[col_v.at[j]], gbufs[b],
                                      gsems.at[b]).wait()
                pltpu.async_copy(gbufs[b], acc.at[row_v.at[j]], ssems.at[b],
                                 add=True)

                @pl.when(cid == 1)
                def _(j=j):
                    pltpu.async_copy(ones_v, dacc.at[row_v.at[j]], dsem, add=True)
            return 0
        lax.fori_loop(0, STAGE // RING, _ring, 0)

        for b in range(RING):
            pltpu.make_async_copy(gbufs[b], acc.at[row_v.at[STAGE - RING + b]],
                                  ssems.at[b]).wait()
        return 0
    lax.fori_loop(0, nstages, _stage, 0)

    @pl.when(cid == 1)
    def _():
        def _dr(t, _):
            pltpu.make_async_copy(ones_v, dacc.at[pl.ds(0, CHUNK)], dsem).wait()
            return 0
        lax.fori_loop(0, nstages * STAGE, _dr, 0)

    plsc.subcore_barrier()

    @pl.when((cid == 0) & jnp.logical_not(last))
    def _():
        pltpu.sync_copy(acc.at[pl.ds(nbase, NSPLIT)], s1_hbm.at[pl.ds(nbase, NSPLIT)])

    @pl.when((cid == 0) & last)
    def _():
        pltpu.sync_copy(acc.at[pl.ds(nbase, NLAST)], s1_hbm.at[pl.ds(nbase, NLAST)])

    @pl.when((cid == 1) & jnp.logical_not(last))
    def _():
        pltpu.sync_copy(acc.at[pl.ds(nbase, NSPLIT)], s2_hbm.at[pl.ds(nbase, NSPLIT)])
        pltpu.sync_copy(dacc.at[pl.ds(nbase, NSPLIT)], deg_hbm.at[pl.ds(nbase, NSPLIT)])

    @pl.when((cid == 1) & last)
    def _():
        pltpu.sync_copy(acc.at[pl.ds(nbase, NLAST)], s2_hbm.at[pl.ds(nbase, NLAST)])
        pltpu.sync_copy(dacc.at[pl.ds(nbase, NLAST)], deg_hbm.at[pl.ds(nbase, NLAST)])


_seg_kernel = functools.partial(
    pl.kernel,
    out_type=(jax.ShapeDtypeStruct((N, D), jnp.float32),
              jax.ShapeDtypeStruct((N, D), jnp.float32),
              jax.ShapeDtypeStruct((N, 8), jnp.float32)),
    mesh=plsc.VectorSubcoreMesh(core_axis_name="c", subcore_axis_name="s",
                                num_cores=NC, num_subcores=NS),
    compiler_params=pltpu.CompilerParams(use_tc_tiling_on_sc=False),
    scratch_types=[
        pltpu.VMEM((STAGE, CHUNK), jnp.int32),
        pltpu.VMEM((STAGE, CHUNK), jnp.int32),
        pltpu.VMEM((CHUNK, D), jnp.float32),
        pltpu.VMEM((CHUNK, D), jnp.float32),
        pltpu.VMEM((CHUNK, 8), jnp.float32),
        pltpu.VMEM((CHUNK, 8), jnp.float32),
        pltpu.VMEM_SHARED((NACC, D), jnp.float32),
        pltpu.VMEM_SHARED((NACC, 8), jnp.float32),
        pltpu.SemaphoreType.DMA((RING,)),
        pltpu.SemaphoreType.DMA((RING,)),
        pltpu.SemaphoreType.DMA,
    ],
)(_seg_body)


def _epi_body(avg_ref, s1_ref, s2_ref, deg_ref, x_ref,
              wpre_ref, wlin_ref, bias_ref, out_ref):
    avg = avg_ref[0, 0]
    s1 = s1_ref[...]
    s2 = s2_ref[...]
    x = x_ref[...]
    std = jnp.sqrt(jnp.maximum(s2 - s1 * s1, 0.0) + EPS)
    logd = jnp.log(deg_ref[:, 0:1] + 1.0)
    amp = logd / avg
    att = avg / (logd + EPS)

    def dot_t(a, w):
        return lax.dot_general(a, w, (((1,), (1,)), ((), ())),
                               preferred_element_type=jnp.float32)

    h_id = dot_t(s1, wpre_ref[0]) + dot_t(std, wpre_ref[3]) + dot_t(x, wlin_ref[...])
    h_amp = dot_t(s1, wpre_ref[1]) + dot_t(std, wpre_ref[4])
    h_att = dot_t(s1, wpre_ref[2]) + dot_t(std, wpre_ref[5])
    out_ref[...] = h_id + amp * h_amp + att * h_att + bias_ref[...]


def _epilogue(avg, s1, s2, deg, x, w_pre, w_lin, bias):
    blk = 2000
    return pl.pallas_call(
        _epi_body,
        grid=(N // blk,),
        in_specs=[
            pl.BlockSpec(memory_space=pltpu.SMEM),
            pl.BlockSpec((blk, D), lambda i: (i, 0)),
            pl.BlockSpec((blk, D), lambda i: (i, 0)),
            pl.BlockSpec((blk, 8), lambda i: (i, 0)),
            pl.BlockSpec((blk, D), lambda i: (i, 0)),
            pl.BlockSpec((N_HEADS, OUT, D), lambda i: (0, 0, 0)),
            pl.BlockSpec((OUT, D), lambda i: (0, 0)),
            pl.BlockSpec((1, OUT), lambda i: (0, 0)),
        ],
        out_specs=pl.BlockSpec((blk, OUT), lambda i: (i, 0)),
        out_shape=jax.ShapeDtypeStruct((N, OUT), jnp.float32),
    )(avg, s1, s2, deg, x, w_pre, w_lin, bias)


def kernel(x, edge_index, avg_deg_log, W_pre, W_lin, bias):
    npadE = EP - E
    padk = jnp.arange(npadE, dtype=jnp.int32)
    row_p = jnp.concatenate([edge_index[0], N + padk % NPAD])
    col_p = jnp.concatenate([edge_index[1], padk % N])
    row2 = row_p.reshape(NROWS, CHUNK)
    col2 = col_p.reshape(NROWS, CHUNK)
    colN2 = col2 + N

    xs = _square_stack(x).reshape(2 * N, D)
    s1, s2, deg = _seg_kernel(xs, row2, col2, colN2)

    avg = jnp.reshape(avg_deg_log, (1, 1))
    return _epilogue(avg, s1, s2, deg, x, W_pre, W_lin, bias.reshape(1, OUT))


# speedup vs baseline: 1.2070x; 1.2070x over previous
import functools

import jax
import jax.numpy as jnp
from jax import lax
from jax.experimental import pallas as pl
from jax.experimental.pallas import tpu as pltpu
from jax.experimental.pallas import tpu_sc as plsc

N = 10000
E = 320000
D = 128
OUT = 128
EPS = 1e-07
N_HEADS = 6

NC = 2
NS = 16
CHUNK = 64
NROWS = E // CHUNK
ROWS_TILE = 320
ROWS_LAST = NROWS - ROWS_TILE * (NS - 1)
NSPLIT = 640
NLAST = N - NSPLIT * (NS - 1)
STAGE = 40
RING = 4


def _square_stack_body(x_ref, out_ref):
    xb = x_ref[...]
    out_ref[0] = xb
    out_ref[1] = xb * xb


def _square_stack(x):
    blk = 2000
    return pl.pallas_call(
        _square_stack_body,
        grid=(N // blk,),
        in_specs=[pl.BlockSpec((blk, D), lambda i: (i, 0))],
        out_specs=pl.BlockSpec((2, blk, D), lambda i: (0, i, 0)),
        out_shape=jax.ShapeDtypeStruct((2, N, D), jnp.float32),
    )(x)


def _seg_body(xs_hbm, row_hbm, col_hbm, colN_hbm, s1_hbm, s2_hbm,
              deg_hbm, row_v, col_v, gb0, gb1, gb2, gb3, zb, ones_v, acc, dacc,
              gsems, ssems, dsem, isem):
    cid = lax.axis_index("c")
    sid = lax.axis_index("s")
    last = sid == NS - 1
    gbufs = (gb0, gb1, gb2, gb3)

    zero16 = jnp.zeros((16,), jnp.float32)
    one16 = jnp.full((16,), 1.0, jnp.float32)

    def _zg(t, _):
        r = t // (D // 16)
        k = (t % (D // 16)) * 16
        gb0[r, pl.ds(k, 16)] = zero16
        gb1[r, pl.ds(k, 16)] = one16
        return 0
    lax.fori_loop(0, CHUNK * (D // 16), _zg, 0)

    stg = sid * 2 * CHUNK
    pltpu.sync_copy(gb0.at[pl.ds(0, CHUNK), pl.ds(0, 8)], dacc.at[pl.ds(stg, CHUNK)])
    pltpu.sync_copy(gb1.at[pl.ds(0, CHUNK), pl.ds(0, 8)],
                    dacc.at[pl.ds(stg + CHUNK, CHUNK)])
    pltpu.sync_copy(dacc.at[pl.ds(stg, CHUNK)], zb)
    pltpu.sync_copy(dacc.at[pl.ds(stg + CHUNK, CHUNK)], ones_v)

    nbase = sid * NSPLIT
    nz = jnp.where(last, NLAST // CHUNK, NSPLIT // CHUNK)

    def _zacc(t, _):
        pltpu.sync_copy(gb0, acc.at[pl.ds(nbase + t * CHUNK, CHUNK)])
        pltpu.sync_copy(zb, dacc.at[pl.ds(nbase + t * CHUNK, CHUNK)])
        return 0
    lax.fori_loop(0, nz, _zacc, 0)

    @pl.when(last)
    def _():
        tb = nbase + (NLAST // CHUNK) * CHUNK
        pltpu.sync_copy(gb0.at[pl.ds(0, NLAST % CHUNK)], acc.at[pl.ds(tb, NLAST % CHUNK)])
        pltpu.sync_copy(zb.at[pl.ds(0, NLAST % CHUNK)], dacc.at[pl.ds(tb, NLAST % CHUNK)])

    plsc.subcore_barrier()

    nstages = jnp.where(last, ROWS_LAST // STAGE, ROWS_TILE // STAGE)
    ips = STAGE // RING

    def _load_idx(st, p, sync):
        rb = sid * ROWS_TILE + st * STAGE

        @pl.when(cid == 0)
        def _():
            if sync:
                pltpu.sync_copy(col_hbm.at[pl.ds(rb, STAGE)], col_v.at[p])
            else:
                pltpu.async_copy(col_hbm.at[pl.ds(rb, STAGE)], col_v.at[p], isem)

        @pl.when(cid == 1)
        def _():
            if sync:
                pltpu.sync_copy(colN_hbm.at[pl.ds(rb, STAGE)], col_v.at[p])
            else:
                pltpu.async_copy(colN_hbm.at[pl.ds(rb, STAGE)], col_v.at[p], isem)
        if sync:
            pltpu.sync_copy(row_hbm.at[pl.ds(rb, STAGE)], row_v.at[p])
        else:
            pltpu.async_copy(row_hbm.at[pl.ds(rb, STAGE)], row_v.at[p], isem)

    _load_idx(0, 0, True)
    _load_idx(1, 1, False)

    def _ring(kk, _):
        st = kk // ips
        p = st % 2
        jb = (kk % ips) * RING
        boundary = (kk % ips == 0) & (st > 0)

        @pl.when(boundary)
        def _():
            pltpu.make_async_copy(row_hbm.at[pl.ds(0, STAGE)], row_v.at[p],
                                  isem).wait()
            pltpu.make_async_copy(row_hbm.at[pl.ds(0, STAGE)], col_v.at[p],
                                  isem).wait()

        for b in range(RING):
            @pl.when(kk > 0)
            def _(b=b):
                pltpu.make_async_copy(gbufs[b], acc.at[row_v.at[p, jb + b]],
                                      ssems.at[b]).wait()
            pltpu.async_copy(xs_hbm.at[col_v.at[p, jb + b]], gbufs[b],
                             gsems.at[b])

        @pl.when(boundary)
        def _():
            @pl.when(cid == 1)
            def _():
                def _dr(t, _):
                    pltpu.make_async_copy(ones_v, dacc.at[pl.ds(0, CHUNK)],
                                          dsem).wait()
                    return 0
                lax.fori_loop(0, STAGE, _dr, 0)

            @pl.when(st + 1 < nstages)
            def _():
                _load_idx(st + 1, 1 - p, False)

        for b in range(RING):
            pltpu.make_async_copy(xs_hbm.at[col_v.at[p, jb + b]], gbufs[b],
                                  gsems.at[b]).wait()
            pltpu.async_copy(gbufs[b], acc.at[row_v.at[p, jb + b]], ssems.at[b],
                             add=True)

            @pl.when(cid == 1)
            def _(b=b):
                pltpu.async_copy(ones_v, dacc.at[row_v.at[p, jb + b]], dsem,
                                 add=True)
        return 0
    lax.fori_loop(0, nstages * ips, _ring, 0)

    plast = (nstages - 1) % 2
    for b in range(RING):
        pltpu.make_async_copy(gbufs[b], acc.at[row_v.at[plast, STAGE - RING + b]],
                              ssems.at[b]).wait()

    @pl.when(cid == 1)
    def _():
        def _dr(t, _):
            pltpu.make_async_copy(ones_v, dacc.at[pl.ds(0, CHUNK)], dsem).wait()
            return 0
        lax.fori_loop(0, STAGE, _dr, 0)

    plsc.subcore_barrier()

    @pl.when((cid == 0) & jnp.logical_not(last))
    def _():
        pltpu.sync_copy(acc.at[pl.ds(nbase, NSPLIT)], s1_hbm.at[pl.ds(nbase, NSPLIT)])

    @pl.when((cid == 0) & last)
    def _():
        pltpu.sync_copy(acc.at[pl.ds(nbase, NLAST)], s1_hbm.at[pl.ds(nbase, NLAST)])

    @pl.when((cid == 1) & jnp.logical_not(last))
    def _():
        pltpu.sync_copy(acc.at[pl.ds(nbase, NSPLIT)], s2_hbm.at[pl.ds(nbase, NSPLIT)])
        pltpu.sync_copy(dacc.at[pl.ds(nbase, NSPLIT)], deg_hbm.at[pl.ds(nbase, NSPLIT)])

    @pl.when((cid == 1) & last)
    def _():
        pltpu.sync_copy(acc.at[pl.ds(nbase, NLAST)], s2_hbm.at[pl.ds(nbase, NLAST)])
        pltpu.sync_copy(dacc.at[pl.ds(nbase, NLAST)], deg_hbm.at[pl.ds(nbase, NLAST)])


_seg_kernel = functools.partial(
    pl.kernel,
    out_type=(jax.ShapeDtypeStruct((N, D), jnp.float32),
              jax.ShapeDtypeStruct((N, D), jnp.float32),
              jax.ShapeDtypeStruct((N, 8), jnp.float32)),
    mesh=plsc.VectorSubcoreMesh(core_axis_name="c", subcore_axis_name="s",
                                num_cores=NC, num_subcores=NS),
    compiler_params=pltpu.CompilerParams(use_tc_tiling_on_sc=False),
    scratch_types=[
        pltpu.VMEM((2, STAGE, CHUNK), jnp.int32),
        pltpu.VMEM((2, STAGE, CHUNK), jnp.int32),
        pltpu.VMEM((CHUNK, D), jnp.float32),
        pltpu.VMEM((CHUNK, D), jnp.float32),
        pltpu.VMEM((CHUNK, D), jnp.float32),
        pltpu.VMEM((CHUNK, D), jnp.float32),
        pltpu.VMEM((CHUNK, 8), jnp.float32),
        pltpu.VMEM((CHUNK, 8), jnp.float32),
        pltpu.VMEM_SHARED((N, D), jnp.float32),
        pltpu.VMEM_SHARED((N, 8), jnp.float32),
        pltpu.SemaphoreType.DMA((RING,)),
        pltpu.SemaphoreType.DMA((RING,)),
        pltpu.SemaphoreType.DMA,
        pltpu.SemaphoreType.DMA,
    ],
)(_seg_body)


def _epi_body(avg_ref, s1_ref, s2_ref, deg_ref, x_ref,
              wpre_ref, wlin_ref, bias_ref, out_ref):
    avg = avg_ref[0, 0]
    s1 = s1_ref[...]
    s2 = s2_ref[...]
    x = x_ref[...]
    std = jnp.sqrt(jnp.maximum(s2 - s1 * s1, 0.0) + EPS)
    logd = jnp.log(deg_ref[:, 0:1] + 1.0)
    amp = logd / avg
    att = avg / (logd + EPS)

    def dot_t(a, w):
        return lax.dot_general(a.astype(jnp.bfloat16), w.astype(jnp.bfloat16),
                               (((1,), (1,)), ((), ())),
                               preferred_element_type=jnp.float32)

    h_id = dot_t(s1, wpre_ref[0]) + dot_t(std, wpre_ref[3]) + dot_t(x, wlin_ref[...])
    h_amp = dot_t(s1, wpre_ref[1]) + dot_t(std, wpre_ref[4])
    h_att = dot_t(s1, wpre_ref[2]) + dot_t(std, wpre_ref[5])
    out_ref[...] = h_id + amp * h_amp + att * h_att + bias_ref[...]


def _epilogue(avg, s1, s2, deg, x, w_pre, w_lin, bias):
    blk = 2000
    return pl.pallas_call(
        _epi_body,
        grid=(N // blk,),
        in_specs=[
            pl.BlockSpec(memory_space=pltpu.SMEM),
            pl.BlockSpec((blk, D), lambda i: (i, 0)),
            pl.BlockSpec((blk, D), lambda i: (i, 0)),
            pl.BlockSpec((blk, 8), lambda i: (i, 0)),
            pl.BlockSpec((blk, D), lambda i: (i, 0)),
            pl.BlockSpec((N_HEADS, OUT, D), lambda i: (0, 0, 0)),
            pl.BlockSpec((OUT, D), lambda i: (0, 0)),
            pl.BlockSpec((1, OUT), lambda i: (0, 0)),
        ],
        out_specs=pl.BlockSpec((blk, OUT), lambda i: (i, 0)),
        out_shape=jax.ShapeDtypeStruct((N, OUT), jnp.float32),
    )(avg, s1, s2, deg, x, w_pre, w_lin, bias)


def kernel(x, edge_index, avg_deg_log, W_pre, W_lin, bias):
    row2 = edge_index[0].reshape(NROWS, CHUNK)
    col2 = edge_index[1].reshape(NROWS, CHUNK)
    colN2 = col2 + N

    xs = _square_stack(x).reshape(2 * N, D)
    s1, s2, deg = _seg_kernel(xs, row2, col2, colN2)

    avg = jnp.reshape(avg_deg_log, (1, 1))
    return _epilogue(avg, s1, s2, deg, x, W_pre, W_lin, bias.reshape(1, OUT))


# speedup vs baseline: 1.2381x; 1.0258x over previous
import functools

import jax
import jax.numpy as jnp
from jax import lax
from jax.experimental import pallas as pl
from jax.experimental.pallas import tpu as pltpu
from jax.experimental.pallas import tpu_sc as plsc

N = 10000
E = 320000
D = 128
OUT = 128
EPS = 1e-07
N_HEADS = 6

NC = 2
NS = 16
CHUNK = 64
NROWS = E // CHUNK
ROWS_TILE = 320
ROWS_LAST = NROWS - ROWS_TILE * (NS - 1)
NSPLIT = 640
NLAST = N - NSPLIT * (NS - 1)
STAGE = 40
RING = 4


def _square_body(x_ref, out_ref):
    xb = x_ref[...]
    out_ref[...] = xb * xb


def _square(x):
    blk = 2000
    return pl.pallas_call(
        _square_body,
        grid=(N // blk,),
        in_specs=[pl.BlockSpec((blk, D), lambda i: (i, 0))],
        out_specs=pl.BlockSpec((blk, D), lambda i: (i, 0)),
        out_shape=jax.ShapeDtypeStruct((N, D), jnp.float32),
    )(x)


def _seg_body(x_hbm, xsq_hbm, row_hbm, col_hbm, s1_hbm, s2_hbm,
              deg_hbm, row_v, col_v, gb0, gb1, gb2, gb3, zb, ones_v, acc, dacc,
              gsems, ssems, dsem, isem):
    cid = lax.axis_index("c")
    sid = lax.axis_index("s")
    last = sid == NS - 1
    gbufs = (gb0, gb1, gb2, gb3)

    zero16 = jnp.zeros((16,), jnp.float32)
    one16 = jnp.full((16,), 1.0, jnp.float32)

    def _zg(t, _):
        r = t // (D // 16)
        k = (t % (D // 16)) * 16
        gb0[r, pl.ds(k, 16)] = zero16
        gb1[r, pl.ds(k, 16)] = one16
        return 0
    lax.fori_loop(0, CHUNK * (D // 16), _zg, 0)

    stg = sid * 2 * CHUNK
    pltpu.sync_copy(gb0.at[pl.ds(0, CHUNK), pl.ds(0, 8)], dacc.at[pl.ds(stg, CHUNK)])
    pltpu.sync_copy(gb1.at[pl.ds(0, CHUNK), pl.ds(0, 8)],
                    dacc.at[pl.ds(stg + CHUNK, CHUNK)])
    pltpu.sync_copy(dacc.at[pl.ds(stg, CHUNK)], zb)
    pltpu.sync_copy(dacc.at[pl.ds(stg + CHUNK, CHUNK)], ones_v)

    nbase = sid * NSPLIT
    nz = jnp.where(last, NLAST // CHUNK, NSPLIT // CHUNK)

    def _zacc(t, _):
        pltpu.sync_copy(gb0, acc.at[pl.ds(nbase + t * CHUNK, CHUNK)])
        pltpu.sync_copy(zb, dacc.at[pl.ds(nbase + t * CHUNK, CHUNK)])
        return 0
    lax.fori_loop(0, nz, _zacc, 0)

    @pl.when(last)
    def _():
        tb = nbase + (NLAST // CHUNK) * CHUNK
        pltpu.sync_copy(gb0.at[pl.ds(0, NLAST % CHUNK)], acc.at[pl.ds(tb, NLAST % CHUNK)])
        pltpu.sync_copy(zb.at[pl.ds(0, NLAST % CHUNK)], dacc.at[pl.ds(tb, NLAST % CHUNK)])

    plsc.subcore_barrier()

    nstages = jnp.where(last, ROWS_LAST // STAGE, ROWS_TILE // STAGE)
    ips = STAGE // RING

    def _load_idx(st, p, sync):
        rb = sid * ROWS_TILE + st * STAGE
        if sync:
            pltpu.sync_copy(col_hbm.at[pl.ds(rb, STAGE)], col_v.at[p])
            pltpu.sync_copy(row_hbm.at[pl.ds(rb, STAGE)], row_v.at[p])
        else:
            pltpu.async_copy(col_hbm.at[pl.ds(rb, STAGE)], col_v.at[p], isem)
            pltpu.async_copy(row_hbm.at[pl.ds(rb, STAGE)], row_v.at[p], isem)

    def _main(src_hbm, with_deg):
        _load_idx(0, 0, True)
        _load_idx(1, 1, False)

        def _ring(kk, _):
            st = kk // ips
            p = st % 2
            jb = (kk % ips) * RING
            boundary = (kk % ips == 0) & (st > 0)

            @pl.when(boundary)
            def _():
                pltpu.make_async_copy(row_hbm.at[pl.ds(0, STAGE)], row_v.at[p],
                                      isem).wait()
                pltpu.make_async_copy(row_hbm.at[pl.ds(0, STAGE)], col_v.at[p],
                                      isem).wait()

            for b in range(RING):
                @pl.when(kk > 0)
                def _(b=b):
                    pltpu.make_async_copy(gbufs[b], acc.at[row_v.at[p, jb + b]],
                                          ssems.at[b]).wait()
                pltpu.async_copy(src_hbm.at[col_v.at[p, jb + b]], gbufs[b],
                                 gsems.at[b])

            @pl.when(boundary)
            def _():
                if with_deg:
                    def _dr(t, _):
                        pltpu.make_async_copy(ones_v, dacc.at[pl.ds(0, CHUNK)],
                                              dsem).wait()
                        return 0
                    lax.fori_loop(0, STAGE, _dr, 0)

                @pl.when(st + 1 < nstages)
                def _():
                    _load_idx(st + 1, 1 - p, False)

            for b in range(RING):
                pltpu.make_async_copy(src_hbm.at[col_v.at[p, jb + b]], gbufs[b],
                                      gsems.at[b]).wait()
                pltpu.async_copy(gbufs[b], acc.at[row_v.at[p, jb + b]],
                                 ssems.at[b], add=True)
                if with_deg:
                    pltpu.async_copy(ones_v, dacc.at[row_v.at[p, jb + b]], dsem,
                                     add=True)
            return 0
        lax.fori_loop(0, nstages * ips, _ring, 0)

        plast = (nstages - 1) % 2
        for b in range(RING):
            pltpu.make_async_copy(gbufs[b],
                                  acc.at[row_v.at[plast, STAGE - RING + b]],
                                  ssems.at[b]).wait()
        if with_deg:
            def _dr(t, _):
                pltpu.make_async_copy(ones_v, dacc.at[pl.ds(0, CHUNK)], dsem).wait()
                return 0
            lax.fori_loop(0, STAGE, _dr, 0)

    @pl.when(cid == 0)
    def _():
        _main(x_hbm, False)

    @pl.when(cid == 1)
    def _():
        _main(xsq_hbm, True)

    plsc.subcore_barrier()

    @pl.when((cid == 0) & jnp.logical_not(last))
    def _():
        pltpu.sync_copy(acc.at[pl.ds(nbase, NSPLIT)], s1_hbm.at[pl.ds(nbase, NSPLIT)])

    @pl.when((cid == 0) & last)
    def _():
        pltpu.sync_copy(acc.at[pl.ds(nbase, NLAST)], s1_hbm.at[pl.ds(nbase, NLAST)])

    @pl.when((cid == 1) & jnp.logical_not(last))
    def _():
        pltpu.sync_copy(acc.at[pl.ds(nbase, NSPLIT)], s2_hbm.at[pl.ds(nbase, NSPLIT)])
        pltpu.sync_copy(dacc.at[pl.ds(nbase, NSPLIT)], deg_hbm.at[pl.ds(nbase, NSPLIT)])

    @pl.when((cid == 1) & last)
    def _():
        pltpu.sync_copy(acc.at[pl.ds(nbase, NLAST)], s2_hbm.at[pl.ds(nbase, NLAST)])
        pltpu.sync_copy(dacc.at[pl.ds(nbase, NLAST)], deg_hbm.at[pl.ds(nbase, NLAST)])


_seg_kernel = functools.partial(
    pl.kernel,
    out_type=(jax.ShapeDtypeStruct((N, D), jnp.float32),
              jax.ShapeDtypeStruct((N, D), jnp.float32),
              jax.ShapeDtypeStruct((N, 8), jnp.float32)),
    mesh=plsc.VectorSubcoreMesh(core_axis_name="c", subcore_axis_name="s",
                                num_cores=NC, num_subcores=NS),
    compiler_params=pltpu.CompilerParams(use_tc_tiling_on_sc=False),
    scratch_types=[
        pltpu.VMEM((2, STAGE, CHUNK), jnp.int32),
        pltpu.VMEM((2, STAGE, CHUNK), jnp.int32),
        pltpu.VMEM((CHUNK, D), jnp.float32),
        pltpu.VMEM((CHUNK, D), jnp.float32),
        pltpu.VMEM((CHUNK, D), jnp.float32),
        pltpu.VMEM((CHUNK, D), jnp.float32),
        pltpu.VMEM((CHUNK, 8), jnp.float32),
        pltpu.VMEM((CHUNK, 8), jnp.float32),
        pltpu.VMEM_SHARED((N, D), jnp.float32),
        pltpu.VMEM_SHARED((N, 8), jnp.float32),
        pltpu.SemaphoreType.DMA((RING,)),
        pltpu.SemaphoreType.DMA((RING,)),
        pltpu.SemaphoreType.DMA,
        pltpu.SemaphoreType.DMA,
    ],
)(_seg_body)


def _epi_body(avg_ref, s1_ref, s2_ref, deg_ref, x_ref,
              wpre_ref, wlin_ref, bias_ref, out_ref):
    avg = avg_ref[0, 0]
    s1 = s1_ref[...]
    s2 = s2_ref[...]
    x = x_ref[...]
    std = jnp.sqrt(jnp.maximum(s2 - s1 * s1, 0.0) + EPS)
    logd = jnp.log(deg_ref[:, 0:1] + 1.0)
    amp = logd / avg
    att = avg / (logd + EPS)

    def dot_t(a, w):
        return lax.dot_general(a.astype(jnp.bfloat16), w.astype(jnp.bfloat16),
                               (((1,), (1,)), ((), ())),
                               preferred_element_type=jnp.float32)

    h_id = dot_t(s1, wpre_ref[0]) + dot_t(std, wpre_ref[3]) + dot_t(x, wlin_ref[...])
    h_amp = dot_t(s1, wpre_ref[1]) + dot_t(std, wpre_ref[4])
    h_att = dot_t(s1, wpre_ref[2]) + dot_t(std, wpre_ref[5])
    out_ref[...] = h_id + amp * h_amp + att * h_att + bias_ref[...]


def _epilogue(avg, s1, s2, deg, x, w_pre, w_lin, bias):
    blk = 2000
    return pl.pallas_call(
        _epi_body,
        grid=(N // blk,),
        in_specs=[
            pl.BlockSpec(memory_space=pltpu.SMEM),
            pl.BlockSpec((blk, D), lambda i: (i, 0)),
            pl.BlockSpec((blk, D), lambda i: (i, 0)),
            pl.BlockSpec((blk, 8), lambda i: (i, 0)),
            pl.BlockSpec((blk, D), lambda i: (i, 0)),
            pl.BlockSpec((N_HEADS, OUT, D), lambda i: (0, 0, 0)),
            pl.BlockSpec((OUT, D), lambda i: (0, 0)),
            pl.BlockSpec((1, OUT), lambda i: (0, 0)),
        ],
        out_specs=pl.BlockSpec((blk, OUT), lambda i: (i, 0)),
        out_shape=jax.ShapeDtypeStruct((N, OUT), jnp.float32),
    )(avg, s1, s2, deg, x, w_pre, w_lin, bias)


def kernel(x, edge_index, avg_deg_log, W_pre, W_lin, bias):
    row2 = edge_index[0].reshape(NROWS, CHUNK)
    col2 = edge_index[1].reshape(NROWS, CHUNK)

    xsq = _square(x)
    s1, s2, deg = _seg_kernel(x, xsq, row2, col2)

    avg = jnp.reshape(avg_deg_log, (1, 1))
    return _epilogue(avg, s1, s2, deg, x, W_pre, W_lin, bias.reshape(1, OUT))
